# baseline (speedup 1.0000x reference)
import functools

import jax
import jax.numpy as jnp
from jax import lax
from jax.experimental import pallas as pl
from jax.experimental.pallas import tpu as pltpu
from jax.experimental.pallas import tpu_sc as plsc

N = 10000
E = 320000
NC, NS, L = 2, 16, 16
NW = NC * NS
EPW = E // NW
C = 128
NCH = EPW // C
TAIL = EPW - NCH * C
RPT8 = 624
NTAIL = N - NS * RPT8
F = 64
RB = 1000
_mesh = plsc.VectorSubcoreMesh(
    core_axis_name="c", subcore_axis_name="s", num_cores=NC, num_subcores=NS)


PR = 128
RPP = PR // NS
EPS = E // NS


def _sc_degree_dis(dst, zeros2d):
    @functools.partial(
        pl.kernel,
        out_type=jax.ShapeDtypeStruct((PR, 128), jnp.float32),
        mesh=_mesh,
        scratch_types=[
            pltpu.VMEM((EPS,), jnp.int32),
            pltpu.VMEM((PR, 128), jnp.float32),
            pltpu.VMEM((PR,), jnp.int32),
            pltpu.VMEM((RPP, 128), jnp.float32),
            pltpu.VMEM_SHARED((PR, 128), jnp.float32),
        ],
        compiler_params=pltpu.CompilerParams(
            needs_layout_passes=False, use_tc_tiling_on_sc=False),
    )
    def body(dst_hbm, zeros_hbm, out_hbm, idx_v, hist_v, rowids, dbuf, acc):
        cid = lax.axis_index("c")
        sid = lax.axis_index("s")
        pltpu.sync_copy(zeros_hbm, hist_v)
        pltpu.sync_copy(zeros_hbm.at[pl.ds(0, RPP)], acc.at[pl.ds(sid * RPP, RPP)])
        pltpu.sync_copy(dst_hbm.at[pl.ds(sid * EPS, EPS)], idx_v)
        for j in range(PR // L):
            rowids[pl.ds(j * L, L)] = lax.broadcasted_iota(jnp.int32, (L,), 0) + j * L
        ones = jnp.ones((L,), jnp.float32)

        def step(j, carry):
            idx = idx_v[pl.ds(j * L, L)]
            row = lax.shift_right_logical(idx, 7)
            col = lax.bitwise_and(idx, 127)
            plsc.addupdate_scatter(hist_v, [row, col], ones)
            return carry

        lax.fori_loop(0, EPS // L, step, 0)
        plsc.subcore_barrier()
        pltpu.sync_copy(hist_v, acc.at[rowids], add=True)
        plsc.subcore_barrier()
        pltpu.sync_copy(acc.at[pl.ds(sid * RPP, RPP)], dbuf)
        for r in range(RPP):
            for j in range(128 // L):
                d = dbuf[r, pl.ds(j * L, L)] + 1.0
                xi = lax.bitcast_convert_type(d, jnp.int32)
                yi = jnp.int32(0x5F3759DF) - lax.shift_right_arithmetic(xi, 1)
                y = lax.bitcast_convert_type(yi, jnp.float32)
                for _ in range(3):
                    y = y * (1.5 - 0.5 * d * y * y)
                dbuf[r, pl.ds(j * L, L)] = y
        mine = jnp.where(cid == 0, sid < NS // 2, sid >= NS // 2)

        @pl.when(mine)
        def _write():
            pltpu.sync_copy(dbuf, out_hbm.at[pl.ds(sid * RPP, RPP)])

    return body(dst, zeros2d)


def _sc_aggregate(yw, src, dst, zeros_t):
    @functools.partial(
        pl.kernel,
        out_type=jax.ShapeDtypeStruct((NC, N, F), jnp.float32),
        mesh=_mesh,
        scratch_types=[
            pltpu.VMEM((EPW,), jnp.int32),
            pltpu.VMEM((EPW,), jnp.int32),
            pltpu.VMEM((C,), jnp.int32),
            pltpu.VMEM((C,), jnp.int32),
            pltpu.VMEM((C,), jnp.int32),
            pltpu.VMEM((C, F), jnp.float32),
            pltpu.VMEM((C, F), jnp.float32),
            pltpu.VMEM((C, F), jnp.float32),
            pltpu.VMEM((TAIL,), jnp.int32),
            pltpu.VMEM((TAIL, F), jnp.float32),
            pltpu.VMEM_SHARED((N, F), jnp.float32),
            pltpu.SemaphoreType.DMA,
            pltpu.SemaphoreType.DMA,
            pltpu.SemaphoreType.DMA,
            pltpu.SemaphoreType.DMA,
            pltpu.SemaphoreType.DMA,
            pltpu.SemaphoreType.DMA,
        ],
        compiler_params=pltpu.CompilerParams(
            needs_layout_passes=False, use_tc_tiling_on_sc=False),
    )
    def body(yw_hbm, src_hbm, dst_hbm, zeros_hbm, out_hbm, sidx, didx,
             di0, di1, di2, r0, r1, r2, dit, rt, acc, g0, g1, g2, s0, s1, s2):
        cid = lax.axis_index("c")
        sid = lax.axis_index("s")
        wid = sid * NC + cid
        dis_b = (di0, di1, di2)
        row_b = (r0, r1, r2)
        gs = (g0, g1, g2)
        ss = (s0, s1, s2)
        pltpu.sync_copy(zeros_hbm, acc.at[pl.ds(sid * RPT8, RPT8)])

        @pl.when(sid == 0)
        def _zero_tail():
            pltpu.sync_copy(zeros_hbm.at[pl.ds(0, NTAIL)],
                            acc.at[pl.ds(NS * RPT8, NTAIL)])

        plsc.subcore_barrier()

        base = wid * EPW
        pltpu.sync_copy(src_hbm.at[pl.ds(base, EPW)], sidx)
        pltpu.sync_copy(dst_hbm.at[pl.ds(base, EPW)], didx)

        def build_di(m, dib):
            for j in range(C // L):
                dib[pl.ds(j * L, L)] = didx[pl.ds(m * C + j * L, L)]

        def start_gather(m, rb, gb):
            pltpu.async_copy(yw_hbm.at[sidx.at[pl.ds(m * C, C)]], rb, gb)

        def wait_gather(m, rb, gb):
            pltpu.make_async_copy(yw_hbm.at[sidx.at[pl.ds(m * C, C)]], rb, gb).wait()

        build_di(0, di0)
        start_gather(0, r0, g0)
        build_di(1, di1)
        start_gather(1, r1, g1)

        def triple(i, carry):
            for b in range(3):
                k = i * 3 + b
                bm1 = (b + 2) % 3
                wait_gather(k, row_b[b], gs[b])
                pltpu.async_copy(row_b[b], acc.at[dis_b[b]], ss[b], add=True)

                @pl.when(k >= 1)
                def _wait_prev_scatter():
                    pltpu.make_async_copy(
                        row_b[bm1], acc.at[dis_b[bm1]], ss[bm1]).wait()

                @pl.when(k + 2 < NCH)
                def _prefetch():
                    build_di(k + 2, dis_b[bm1])
                    start_gather(k + 2, row_b[bm1], gs[bm1])

            return carry

        lax.fori_loop(0, NCH // 3, triple, 0)
        _blast = (NCH - 1) % 3
        pltpu.make_async_copy(row_b[_blast], acc.at[dis_b[_blast]], ss[_blast]).wait()
        for j in range(TAIL // L):
            dit[pl.ds(j * L, L)] = didx[pl.ds(NCH * C + j * L, L)]
        pltpu.async_copy(yw_hbm.at[sidx.at[pl.ds(NCH * C, TAIL)]], rt, g0).wait()
        pltpu.sync_copy(rt, acc.at[dit], add=True)
        plsc.subcore_barrier()
        pltpu.sync_copy(acc.at[pl.ds(sid * RPT8, RPT8)],
                        out_hbm.at[cid, pl.ds(sid * RPT8, RPT8)])

        @pl.when(sid == 0)
        def _write_tail():
            pltpu.sync_copy(acc.at[pl.ds(NS * RPT8, NTAIL)],
                            out_hbm.at[cid, pl.ds(NS * RPT8, NTAIL)])

    return body(yw, src, dst, zeros_t)


def _tc_prep(x, W1, dis):
    def body(x_ref, w_ref, dis_ref, yw_ref):
        xw = jnp.dot(x_ref[...], w_ref[...], preferred_element_type=jnp.float32)
        yw_ref[...] = xw * dis_ref[...]

    return pl.pallas_call(
        body,
        grid=(N // RB,),
        in_specs=[
            pl.BlockSpec((RB, 128), lambda i: (i, 0)),
            pl.BlockSpec((128, F), lambda i: (0, 0)),
            pl.BlockSpec((RB, 1), lambda i: (i, 0)),
        ],
        out_specs=pl.BlockSpec((RB, F), lambda i: (i, 0)),
        out_shape=jax.ShapeDtypeStruct((N, F), jnp.float32),
    )(x, W1, dis)


def _tc_layer(p, yw_prev, dis, b, W_next):
    def body(p_ref, yw_ref, dis_ref, b_ref, w_ref, out_ref):
        dis = dis_ref[...]
        agg = jnp.sum(p_ref[...], axis=0) + yw_ref[...]
        h = jnp.maximum(agg * dis + b_ref[...], 0.0)
        out_ref[...] = jnp.dot(h, w_ref[...],
                               preferred_element_type=jnp.float32) * dis

    return pl.pallas_call(
        body,
        grid=(N // RB,),
        in_specs=[
            pl.BlockSpec((NC, RB, F), lambda i: (0, i, 0)),
            pl.BlockSpec((RB, F), lambda i: (i, 0)),
            pl.BlockSpec((RB, 1), lambda i: (i, 0)),
            pl.BlockSpec((1, F), lambda i: (0, 0)),
            pl.BlockSpec((F, F), lambda i: (0, 0)),
        ],
        out_specs=pl.BlockSpec((RB, F), lambda i: (i, 0)),
        out_shape=jax.ShapeDtypeStruct((N, F), jnp.float32),
    )(p, yw_prev, dis, b.reshape(1, F), W_next)


def _tc_final(p, yw_prev, dis, b, Wl, bl):
    def body(p_ref, yw_ref, dis_ref, b_ref, wl_ref, bl_ref, h_ref, z_ref):
        dis = dis_ref[...]
        agg = jnp.sum(p_ref[...], axis=0) + yw_ref[...]
        h = jnp.maximum(agg * dis + b_ref[...], 0.0)
        h_ref[...] = h
        z_ref[...] = jnp.dot(h, wl_ref[...],
                             preferred_element_type=jnp.float32) + bl_ref[...]

    return pl.pallas_call(
        body,
        grid=(N // RB,),
        in_specs=[
            pl.BlockSpec((NC, RB, F), lambda i: (0, i, 0)),
            pl.BlockSpec((RB, F), lambda i: (i, 0)),
            pl.BlockSpec((RB, 1), lambda i: (i, 0)),
            pl.BlockSpec((1, F), lambda i: (0, 0)),
            pl.BlockSpec((F, 4), lambda i: (0, 0)),
            pl.BlockSpec((1, 4), lambda i: (0, 0)),
        ],
        out_specs=[
            pl.BlockSpec((RB, F), lambda i: (i, 0)),
            pl.BlockSpec((RB, 4), lambda i: (i, 0)),
        ],
        out_shape=[
            jax.ShapeDtypeStruct((N, F), jnp.float32),
            jax.ShapeDtypeStruct((N, 4), jnp.float32),
        ],
    )(p, yw_prev, dis, b.reshape(1, F), Wl, bl.reshape(1, 4))


def kernel(x, W1, b1, W2, b2, Wl, bl, edges):
    src = edges[0]
    dst = edges[1]
    zeros2d = jnp.zeros((PR, 128), jnp.float32)
    zeros_t = jnp.zeros((RPT8, F), jnp.float32)

    dis2d = _sc_degree_dis(dst, zeros2d)
    dis = dis2d.reshape(-1)[:N, None]
    yw1 = _tc_prep(x, W1, dis)
    p1 = _sc_aggregate(yw1, src, dst, zeros_t)
    yw2 = _tc_layer(p1, yw1, dis, b1, W2)
    p2 = _sc_aggregate(yw2, src, dst, zeros_t)
    h, z = _tc_final(p2, yw2, dis, b2, Wl, bl)
    return (h, z)


# speedup vs baseline: 1.0404x; 1.0404x over previous
import functools

import jax
import jax.numpy as jnp
from jax import lax
from jax.experimental import pallas as pl
from jax.experimental.pallas import tpu as pltpu
from jax.experimental.pallas import tpu_sc as plsc

N = 10000
E = 320000
NC, NS, L = 2, 16, 16
NW = NC * NS
EPW = E // NW
C = 128
NCH = EPW // C
TAIL = EPW - NCH * C
RPT8 = 624
NTAIL = N - NS * RPT8
F = 64
RB = 1000
_mesh = plsc.VectorSubcoreMesh(
    core_axis_name="c", subcore_axis_name="s", num_cores=NC, num_subcores=NS)


def _sc_degree(dst, zeros_n):
    @functools.partial(
        pl.kernel,
        out_type=jax.ShapeDtypeStruct((NW, N), jnp.float32),
        mesh=_mesh,
        scratch_types=[
            pltpu.VMEM((EPW,), jnp.int32),
            pltpu.VMEM((N,), jnp.float32),
        ],
        compiler_params=pltpu.CompilerParams(needs_layout_passes=False),
    )
    def body(dst_hbm, zeros_hbm, out_hbm, idx_v, hist_v):
        cid = lax.axis_index("c")
        sid = lax.axis_index("s")
        wid = sid * NC + cid
        pltpu.sync_copy(zeros_hbm, hist_v)
        pltpu.sync_copy(dst_hbm.at[pl.ds(wid * EPW, EPW)], idx_v)
        ones = jnp.ones((L,), jnp.float32)

        def step(j, carry):
            idx = idx_v[pl.ds(j * L, L)]
            plsc.addupdate_scatter(hist_v, [idx], ones)
            return carry

        lax.fori_loop(0, EPW // L, step, 0)
        pltpu.sync_copy(hist_v, out_hbm.at[wid])

    return body(dst, zeros_n)


def _sc_aggregate(yw, src, dst, zeros_t):
    @functools.partial(
        pl.kernel,
        out_type=jax.ShapeDtypeStruct((NC, N, F), jnp.float32),
        mesh=_mesh,
        scratch_types=[
            pltpu.VMEM((EPW,), jnp.int32),
            pltpu.VMEM((EPW,), jnp.int32),
            pltpu.VMEM((C, F), jnp.float32),
            pltpu.VMEM((C, F), jnp.float32),
            pltpu.VMEM((C, F), jnp.float32),
            pltpu.VMEM((TAIL, F), jnp.float32),
            pltpu.VMEM_SHARED((N, F), jnp.float32),
            pltpu.SemaphoreType.DMA,
            pltpu.SemaphoreType.DMA,
            pltpu.SemaphoreType.DMA,
            pltpu.SemaphoreType.DMA,
            pltpu.SemaphoreType.DMA,
            pltpu.SemaphoreType.DMA,
        ],
        compiler_params=pltpu.CompilerParams(
            needs_layout_passes=False, use_tc_tiling_on_sc=False),
    )
    def body(yw_hbm, src_hbm, dst_hbm, zeros_hbm, out_hbm, sidx, didx,
             r0, r1, r2, rt, acc, g0, g1, g2, s0, s1, s2):
        cid = lax.axis_index("c")
        sid = lax.axis_index("s")
        wid = sid * NC + cid
        row_b = (r0, r1, r2)
        gs = (g0, g1, g2)
        ss = (s0, s1, s2)
        pltpu.sync_copy(zeros_hbm, acc.at[pl.ds(sid * RPT8, RPT8)])

        @pl.when(sid == 0)
        def _zero_tail():
            pltpu.sync_copy(zeros_hbm.at[pl.ds(0, NTAIL)],
                            acc.at[pl.ds(NS * RPT8, NTAIL)])

        plsc.subcore_barrier()

        base = wid * EPW
        pltpu.sync_copy(src_hbm.at[pl.ds(base, EPW)], sidx)
        pltpu.sync_copy(dst_hbm.at[pl.ds(base, EPW)], didx)

        def start_gather(m, rb, gb):
            pltpu.async_copy(yw_hbm.at[sidx.at[pl.ds(m * C, C)]], rb, gb)

        def wait_gather(m, rb, gb):
            pltpu.make_async_copy(yw_hbm.at[sidx.at[pl.ds(m * C, C)]], rb, gb).wait()

        def dchunk(m):
            return didx.at[pl.ds(m * C, C)]

        start_gather(0, r0, g0)
        start_gather(1, r1, g1)

        def triple(i, carry):
            for b in range(3):
                k = i * 3 + b
                bm1 = (b + 2) % 3
                wait_gather(k, row_b[b], gs[b])
                pltpu.async_copy(row_b[b], acc.at[dchunk(k)], ss[b], add=True)

                @pl.when(k >= 1)
                def _wait_prev_scatter():
                    pltpu.make_async_copy(
                        row_b[bm1], acc.at[dchunk(k - 1)], ss[bm1]).wait()

                @pl.when(k + 2 < NCH)
                def _prefetch():
                    start_gather(k + 2, row_b[bm1], gs[bm1])

            return carry

        lax.fori_loop(0, NCH // 3, triple, 0)
        _blast = (NCH - 1) % 3
        pltpu.make_async_copy(row_b[_blast], acc.at[dchunk(NCH - 1)], ss[_blast]).wait()
        pltpu.async_copy(yw_hbm.at[sidx.at[pl.ds(NCH * C, TAIL)]], rt, g0).wait()
        pltpu.sync_copy(rt, acc.at[didx.at[pl.ds(NCH * C, TAIL)]], add=True)
        plsc.subcore_barrier()
        pltpu.sync_copy(acc.at[pl.ds(sid * RPT8, RPT8)],
                        out_hbm.at[cid, pl.ds(sid * RPT8, RPT8)])

        @pl.when(sid == 0)
        def _write_tail():
            pltpu.sync_copy(acc.at[pl.ds(NS * RPT8, NTAIL)],
                            out_hbm.at[cid, pl.ds(NS * RPT8, NTAIL)])

    return body(yw, src, dst, zeros_t)


def _tc_dis(degp):
    def body(degp_ref, dis_ref):
        deg = jnp.sum(degp_ref[...], axis=0) + 1.0
        dis_ref[...] = lax.rsqrt(deg)[:, None]

    return pl.pallas_call(
        body,
        out_shape=jax.ShapeDtypeStruct((N, 1), jnp.float32),
    )(degp)


def _tc_prep(x, W1, dis):
    def body(x_ref, w_ref, dis_ref, yw_ref):
        xw = jnp.dot(x_ref[...], w_ref[...], preferred_element_type=jnp.float32)
        yw_ref[...] = xw * dis_ref[...]

    return pl.pallas_call(
        body,
        grid=(N // RB,),
        in_specs=[
            pl.BlockSpec((RB, 128), lambda i: (i, 0)),
            pl.BlockSpec((128, F), lambda i: (0, 0)),
            pl.BlockSpec((RB, 1), lambda i: (i, 0)),
        ],
        out_specs=pl.BlockSpec((RB, F), lambda i: (i, 0)),
        out_shape=jax.ShapeDtypeStruct((N, F), jnp.float32),
    )(x, W1, dis)


def _tc_layer(p, yw_prev, dis, b, W_next):
    def body(p_ref, yw_ref, dis_ref, b_ref, w_ref, out_ref):
        dis = dis_ref[...]
        agg = jnp.sum(p_ref[...], axis=0) + yw_ref[...]
        h = jnp.maximum(agg * dis + b_ref[...], 0.0)
        out_ref[...] = jnp.dot(h, w_ref[...],
                               preferred_element_type=jnp.float32) * dis

    return pl.pallas_call(
        body,
        grid=(N // RB,),
        in_specs=[
            pl.BlockSpec((NC, RB, F), lambda i: (0, i, 0)),
            pl.BlockSpec((RB, F), lambda i: (i, 0)),
            pl.BlockSpec((RB, 1), lambda i: (i, 0)),
            pl.BlockSpec((1, F), lambda i: (0, 0)),
            pl.BlockSpec((F, F), lambda i: (0, 0)),
        ],
        out_specs=pl.BlockSpec((RB, F), lambda i: (i, 0)),
        out_shape=jax.ShapeDtypeStruct((N, F), jnp.float32),
    )(p, yw_prev, dis, b.reshape(1, F), W_next)


def _tc_final(p, yw_prev, dis, b, Wl, bl):
    def body(p_ref, yw_ref, dis_ref, b_ref, wl_ref, bl_ref, h_ref, z_ref):
        dis = dis_ref[...]
        agg = jnp.sum(p_ref[...], axis=0) + yw_ref[...]
        h = jnp.maximum(agg * dis + b_ref[...], 0.0)
        h_ref[...] = h
        z_ref[...] = jnp.dot(h, wl_ref[...],
                             preferred_element_type=jnp.float32) + bl_ref[...]

    return pl.pallas_call(
        body,
        grid=(N // RB,),
        in_specs=[
            pl.BlockSpec((NC, RB, F), lambda i: (0, i, 0)),
            pl.BlockSpec((RB, F), lambda i: (i, 0)),
            pl.BlockSpec((RB, 1), lambda i: (i, 0)),
            pl.BlockSpec((1, F), lambda i: (0, 0)),
            pl.BlockSpec((F, 4), lambda i: (0, 0)),
            pl.BlockSpec((1, 4), lambda i: (0, 0)),
        ],
        out_specs=[
            pl.BlockSpec((RB, F), lambda i: (i, 0)),
            pl.BlockSpec((RB, 4), lambda i: (i, 0)),
        ],
        out_shape=[
            jax.ShapeDtypeStruct((N, F), jnp.float32),
            jax.ShapeDtypeStruct((N, 4), jnp.float32),
        ],
    )(p, yw_prev, dis, b.reshape(1, F), Wl, bl.reshape(1, 4))


def kernel(x, W1, b1, W2, b2, Wl, bl, edges):
    src = edges[0]
    dst = edges[1]
    zeros_n = jnp.zeros((N,), jnp.float32)
    zeros_t = jnp.zeros((RPT8, F), jnp.float32)

    degp = _sc_degree(dst, zeros_n)
    dis = _tc_dis(degp)
    yw1 = _tc_prep(x, W1, dis)
    p1 = _sc_aggregate(yw1, src, dst, zeros_t)
    yw2 = _tc_layer(p1, yw1, dis, b1, W2)
    p2 = _sc_aggregate(yw2, src, dst, zeros_t)
    h, z = _tc_final(p2, yw2, dis, b2, Wl, bl)
    return (h, z)
